# Initial kernel scaffold; baseline (speedup 1.0000x reference)
import jax
import jax.numpy as jnp
from jax.experimental import pallas as pl


def kernel(nodevec1, nodevec2):
    raise NotImplementedError("write your pallas kernel here")


# speedup vs baseline: 17.0745x; 17.0745x over previous
import functools

import jax
import jax.numpy as jnp
from jax import lax
from jax.experimental import pallas as pl

_N = 4096
_K = 128
_TOPK = 32
_BLOCK_ROWS = 256
_CHUNK = 128


def _body(a_ref, b_ref, o_ref):
    m = jnp.dot(a_ref[...], b_ref[...], preferred_element_type=jnp.float32)
    m = jnp.maximum(m, 0.0)
    rows = m.shape[0]
    n = m.shape[1]
    mi = lax.bitcast_convert_type(m, jnp.int32)

    cm = jnp.max(mi.reshape(rows, n // _CHUNK, _CHUNK), axis=2)
    maxbits = jnp.max(cm, axis=1)
    hi0 = maxbits + 1
    lo0 = jnp.min(cm, axis=1)

    def it(_, carry):
        lo, hi = carry
        mid = lo + (hi - lo) // 2
        cnt = jnp.sum((mi >= mid[:, None]).astype(jnp.int32), axis=1)
        ge = cnt >= _TOPK
        return jnp.where(ge, mid, lo), jnp.where(ge, hi, mid)

    lo, _ = lax.fori_loop(0, 31, it, (lo0, hi0))

    keep = mi >= lo[:, None]
    rowmax = lax.bitcast_convert_type(maxbits, jnp.float32)
    z = jnp.where(keep, m, 0.0)
    e = jnp.exp(z - rowmax[:, None])
    s = jnp.sum(e, axis=1)
    o_ref[...] = e / s[:, None]


@jax.jit
def kernel(nodevec1, nodevec2):
    grid = (_N // _BLOCK_ROWS,)
    return pl.pallas_call(
        _body,
        grid=grid,
        in_specs=[
            pl.BlockSpec((_BLOCK_ROWS, _K), lambda i: (i, 0)),
            pl.BlockSpec((_K, _N), lambda i: (0, 0)),
        ],
        out_specs=pl.BlockSpec((_BLOCK_ROWS, _N), lambda i: (i, 0)),
        out_shape=jax.ShapeDtypeStruct((_N, _N), jnp.float32),
    )(nodevec1, nodevec2)


# speedup vs baseline: 19.4464x; 1.1389x over previous
import functools

import jax
import jax.numpy as jnp
from jax import lax
from jax.experimental import pallas as pl

_N = 4096
_K = 128
_TOPK = 32
_BLOCK_ROWS = 512
_CHUNK = 128


def _body(a_ref, b_ref, o_ref):
    m = jnp.dot(a_ref[...], b_ref[...], preferred_element_type=jnp.float32)
    m = jnp.maximum(m, 0.0)
    rows = m.shape[0]
    n = m.shape[1]
    mi = lax.bitcast_convert_type(m, jnp.int32)

    cm = jnp.max(mi.reshape(rows, n // _CHUNK, _CHUNK), axis=2)
    maxbits = jnp.max(cm, axis=1)
    hi0 = maxbits + 1
    lo0 = jnp.min(cm, axis=1)

    def cond(carry):
        lo, hi = carry
        return jnp.max(hi - lo) > 1

    def it(carry):
        lo, hi = carry
        mid = lo + (hi - lo) // 2
        cnt = jnp.sum((mi >= mid[:, None]).astype(jnp.int32), axis=1)
        ge = cnt >= _TOPK
        return jnp.where(ge, mid, lo), jnp.where(ge, hi, mid)

    lo, _ = lax.while_loop(cond, it, (lo0, hi0))

    keep = mi >= lo[:, None]
    rowmax = lax.bitcast_convert_type(maxbits, jnp.float32)
    z = jnp.where(keep, m, 0.0)
    e = jnp.exp(z - rowmax[:, None])
    s = jnp.sum(e, axis=1)
    o_ref[...] = e / s[:, None]


@jax.jit
def kernel(nodevec1, nodevec2):
    grid = (_N // _BLOCK_ROWS,)
    return pl.pallas_call(
        _body,
        grid=grid,
        in_specs=[
            pl.BlockSpec((_BLOCK_ROWS, _K), lambda i: (i, 0)),
            pl.BlockSpec((_K, _N), lambda i: (0, 0)),
        ],
        out_specs=pl.BlockSpec((_BLOCK_ROWS, _N), lambda i: (i, 0)),
        out_shape=jax.ShapeDtypeStruct((_N, _N), jnp.float32),
    )(nodevec1, nodevec2)
